# Initial kernel scaffold; baseline (speedup 1.0000x reference)
import jax
import jax.numpy as jnp
from jax.experimental import pallas as pl


def kernel(a_H, a_A, a_H_init, a_A_init, b, gamma_H, gamma_A, delta_H, delta_A, int_dur, int_match, int_bin, int_state, int_delta, goal_match, goal_bin, goal_state, goal_delta, goal_is_home):
    raise NotImplementedError("write your pallas kernel here")


# speedup vs baseline: 308.1247x; 308.1247x over previous
import functools

import jax
import jax.numpy as jnp
from jax import lax
from jax.experimental import pallas as pl
from jax.experimental.pallas import tpu as pltpu
from jax.experimental.pallas import tpu_sc as plsc

_SIGMA_A = 1.0
_LAMBDA_REG = 0.01

_NC = 2
_NS = 16
_NW = _NC * _NS
_L = 16

_CI = 800
_CG = 800


def _round_up(x, m):
    return (x + m - 1) // m * m


def _body(MP, PW, NCI, PWG, NCG, PWR,
          aH_h, aA_h, aHi_h, aAi_h, cH_h, cA_h, ro_h,
          dur_h, im_h, ib_h, is_h, id_h,
          gm_h, gb_h, gs_h, gd_h, gh_h, gw_h,
          out_h,
          aH_v, aA_v, cH_v, cA_v, ro_v,
          dur_b, mat_b, bin_b, st_b, dl_b,
          gm_b, gbn_b, gst_b, gdl_b, gh_b, gw_b,
          inibH, inibA, accv, sem):
    wid = lax.axis_index("s") * _NC + lax.axis_index("c")

    pltpu.sync_copy(aH_h, aH_v)
    pltpu.sync_copy(aA_h, aA_v)
    pltpu.sync_copy(cH_h, cH_v)
    pltpu.sync_copy(cA_h, cA_v)
    pltpu.sync_copy(ro_h, ro_v)

    acc = jnp.zeros((_L,), jnp.float32)

    rbase = wid * PWR
    pltpu.sync_copy(aHi_h.at[pl.ds(rbase, PWR)], inibH)
    pltpu.sync_copy(aAi_h.at[pl.ds(rbase, PWR)], inibA)

    def reg_body(i, a):
        s = pl.ds(rbase + i * _L, _L)
        sl = pl.ds(i * _L, _L)
        dH = aH_v[s] - inibH[sl]
        dA = aA_v[s] - inibA[sl]
        return a + (0.5 / (_SIGMA_A * _SIGMA_A)) * (dH * dH + dA * dA)

    acc = lax.fori_loop(0, PWR // _L, reg_body, acc)

    r0 = ro_v[pl.ds(0, _L)]
    r1 = ro_v[pl.ds(_L, _L)]
    scale = jnp.where(wid == 0, jnp.float32(_LAMBDA_REG), jnp.float32(0.0))
    acc = acc + scale * (r0 * r0 + r1 * r1)

    ibase = wid * PW

    def int_chunk(c, a):
        off = ibase + c * _CI
        cps = [
            pltpu.async_copy(dur_h.at[pl.ds(off, _CI)], dur_b, sem),
            pltpu.async_copy(im_h.at[pl.ds(off, _CI)], mat_b, sem),
            pltpu.async_copy(ib_h.at[pl.ds(off, _CI)], bin_b, sem),
            pltpu.async_copy(is_h.at[pl.ds(off, _CI)], st_b, sem),
            pltpu.async_copy(id_h.at[pl.ds(off, _CI)], dl_b, sem),
        ]
        for cp in cps:
            cp.wait()

        def vec(i, aa):
            s = pl.ds(i * _L, _L)
            m = mat_b[s]
            k = bin_b[s] * 20 + st_b[s] * 5 + dl_b[s]
            ah = plsc.load_gather(aH_v, [m])
            av = plsc.load_gather(aA_v, [m])
            ch = plsc.load_gather(cH_v, [k])
            ca = plsc.load_gather(cA_v, [k])
            lh = jnp.minimum(jnp.maximum(ah + ch, -20.0), 20.0)
            la = jnp.minimum(jnp.maximum(av + ca, -20.0), 20.0)
            return aa + (jnp.exp(lh) + jnp.exp(la)) * dur_b[s]

        return lax.fori_loop(0, _CI // _L, vec, a)

    acc = lax.fori_loop(0, NCI, int_chunk, acc)

    gbase = wid * PWG

    def goal_chunk(c, a):
        off = gbase + c * _CG
        cps = [
            pltpu.async_copy(gm_h.at[pl.ds(off, _CG)], gm_b, sem),
            pltpu.async_copy(gb_h.at[pl.ds(off, _CG)], gbn_b, sem),
            pltpu.async_copy(gs_h.at[pl.ds(off, _CG)], gst_b, sem),
            pltpu.async_copy(gd_h.at[pl.ds(off, _CG)], gdl_b, sem),
            pltpu.async_copy(gh_h.at[pl.ds(off, _CG)], gh_b, sem),
            pltpu.async_copy(gw_h.at[pl.ds(off, _CG)], gw_b, sem),
        ]
        for cp in cps:
            cp.wait()

        def vec(i, aa):
            s = pl.ds(i * _L, _L)
            m = gm_b[s]
            k = gbn_b[s] * 20 + gst_b[s] * 5 + gdl_b[s]
            lh = plsc.load_gather(aH_v, [m]) + plsc.load_gather(cH_v, [k])
            la = plsc.load_gather(aA_v, [m]) + plsc.load_gather(cA_v, [k])
            h = gh_b[s]
            w = gw_b[s]
            return aa - (h * lh + (w - h) * la)

        return lax.fori_loop(0, _CG // _L, vec, a)

    acc = lax.fori_loop(0, NCG, goal_chunk, acc)

    accv[...] = acc
    pltpu.sync_copy(accv, out_h.at[pl.ds(wid * _L, _L)])


def kernel(a_H, a_A, a_H_init, a_A_init, b, gamma_H, gamma_A, delta_H,
           delta_A, int_dur, int_match, int_bin, int_state, int_delta,
           goal_match, goal_bin, goal_state, goal_delta, goal_is_home):
    f32 = jnp.float32
    i32 = jnp.int32

    M = a_H.shape[0]
    NI = int_dur.shape[0]
    NG = goal_match.shape[0]

    MP = _round_up(M, _NW * _L)
    PWR = MP // _NW
    PW = NI // _NW
    assert PW * _NW == NI and PW % _CI == 0
    NCI = PW // _CI
    NGP = _round_up(NG, _NW * _CG)
    PWG = NGP // _NW
    NCG = PWG // _CG

    zero = jnp.zeros((1,), f32)
    gHf = jnp.concatenate([zero, gamma_H[:1], gamma_H[1:],
                           (gamma_H[0] + gamma_H[1])[None]])
    gAf = jnp.concatenate([zero, gamma_A[:1], gamma_A[1:],
                           (gamma_A[0] + gamma_A[1])[None]])
    dHf = jnp.concatenate([zero, delta_H])
    dAf = jnp.concatenate([zero, delta_A])
    cH = (b[:, None, None] + gHf[None, :, None] + dHf[None, None, :])
    cA = (b[:, None, None] + gAf[None, :, None] + dAf[None, None, :])
    cH = jnp.pad(cH.reshape(-1), (0, 128 - 120))
    cA = jnp.pad(cA.reshape(-1), (0, 128 - 120))
    ro = jnp.pad(jnp.concatenate([b, gamma_H, gamma_A, delta_H, delta_A]),
                 (0, 32 - 18))

    pad_t = (0, MP - M)
    aH_p = jnp.pad(a_H, pad_t)
    aA_p = jnp.pad(a_A, pad_t)
    aHi_p = jnp.pad(a_H_init, pad_t)
    aAi_p = jnp.pad(a_A_init, pad_t)

    pad_g = (0, NGP - NG)
    gm_p = jnp.pad(goal_match.astype(i32), pad_g)
    gb_p = jnp.pad(goal_bin.astype(i32), pad_g)
    gs_p = jnp.pad(goal_state.astype(i32), pad_g)
    gd_p = jnp.pad(goal_delta.astype(i32), pad_g)
    gh_p = jnp.pad(goal_is_home.astype(f32), pad_g)
    gw_p = jnp.pad(jnp.ones((NG,), f32), pad_g)

    mesh = plsc.VectorSubcoreMesh(core_axis_name="c", subcore_axis_name="s",
                                  num_cores=_NC, num_subcores=_NS)
    call = pl.kernel(
        functools.partial(_body, MP, PW, NCI, PWG, NCG, PWR),
        out_type=jax.ShapeDtypeStruct((_NW * _L,), f32),
        mesh=mesh,
        compiler_params=pltpu.CompilerParams(needs_layout_passes=False),
        scratch_types=[
            pltpu.VMEM((MP,), f32),
            pltpu.VMEM((MP,), f32),
            pltpu.VMEM((128,), f32),
            pltpu.VMEM((128,), f32),
            pltpu.VMEM((32,), f32),
            pltpu.VMEM((_CI,), f32),
            pltpu.VMEM((_CI,), i32),
            pltpu.VMEM((_CI,), i32),
            pltpu.VMEM((_CI,), i32),
            pltpu.VMEM((_CI,), i32),
            pltpu.VMEM((_CG,), i32),
            pltpu.VMEM((_CG,), i32),
            pltpu.VMEM((_CG,), i32),
            pltpu.VMEM((_CG,), i32),
            pltpu.VMEM((_CG,), f32),
            pltpu.VMEM((_CG,), f32),
            pltpu.VMEM((PWR,), f32),
            pltpu.VMEM((PWR,), f32),
            pltpu.VMEM((_L,), f32),
            pltpu.SemaphoreType.DMA,
        ],
    )

    out = call(aH_p, aA_p, aHi_p, aAi_p, cH, cA, ro,
               int_dur, int_match.astype(i32), int_bin.astype(i32),
               int_state.astype(i32), int_delta.astype(i32),
               gm_p, gb_p, gs_p, gd_p, gh_p, gw_p)
    return jnp.sum(out)


# speedup vs baseline: 552.9318x; 1.7945x over previous
import functools

import jax
import jax.numpy as jnp
from jax import lax
from jax.experimental import pallas as pl
from jax.experimental.pallas import tpu as pltpu
from jax.experimental.pallas import tpu_sc as plsc

_SIGMA_A = 1.0
_LAMBDA_REG = 0.01

_NC = 2
_NS = 16
_NW = _NC * _NS
_L = 16

_CI = 2000
_CG = 800
_NG_REAL = 150000


def _round_up(x, m):
    return (x + m - 1) // m * m


def _body(PW, NPAIR, PWG, NCG, PWR,
          aH_h, aA_h, aHi_h, aAi_h, cH_h, cA_h, ro_h,
          dur_h, im_h, ib_h, is_h, id_h,
          gm_h, gb_h, gs_h, gd_h, gh_h,
          out_h,
          aH_v, aA_v, cH_v, cA_v, ro_v,
          dur0, mat0, bin0, st0, dl0,
          dur1, mat1, bin1, st1, dl1,
          gm_b, gbn_b, gst_b, gdl_b, gh_b,
          inibH, inibA, accv, semA, semB, semG):
    wid = lax.axis_index("s") * _NC + lax.axis_index("c")

    ibase = wid * PW
    int_hbm = (dur_h, im_h, ib_h, is_h, id_h)
    set0 = (dur0, mat0, bin0, st0, dl0)
    set1 = (dur1, mat1, bin1, st1, dl1)

    def fire(off, bufs, sem):
        for h, b in zip(int_hbm, bufs):
            pltpu.async_copy(h.at[pl.ds(off, _CI)], b, sem)

    def drain(bufs, sem):
        for h, b in zip(int_hbm, bufs):
            pltpu.make_async_copy(h.at[pl.ds(0, _CI)], b, sem).wait()

    fire(ibase, set0, semA)

    pltpu.sync_copy(aH_h, aH_v)
    pltpu.sync_copy(aA_h, aA_v)
    pltpu.sync_copy(cH_h, cH_v)
    pltpu.sync_copy(cA_h, cA_v)
    pltpu.sync_copy(ro_h, ro_v)

    acc = jnp.zeros((_L,), jnp.float32)

    rbase = wid * PWR
    pltpu.sync_copy(aHi_h.at[pl.ds(rbase, PWR)], inibH)
    pltpu.sync_copy(aAi_h.at[pl.ds(rbase, PWR)], inibA)

    def reg_body(i, a):
        s = pl.ds(rbase + i * _L, _L)
        sl = pl.ds(i * _L, _L)
        dH = aH_v[s] - inibH[sl]
        dA = aA_v[s] - inibA[sl]
        return a + (0.5 / (_SIGMA_A * _SIGMA_A)) * (dH * dH + dA * dA)

    acc = lax.fori_loop(0, PWR // _L, reg_body, acc)

    r0 = ro_v[pl.ds(0, _L)]
    r1 = ro_v[pl.ds(_L, _L)]
    scale = jnp.where(wid == 0, jnp.float32(_LAMBDA_REG), jnp.float32(0.0))
    acc = acc + scale * (r0 * r0 + r1 * r1)

    def int_compute(bufs, a):
        dur_b, mat_b, bin_b, st_b, dl_b = bufs

        def vec(i, aa):
            s = pl.ds(i * _L, _L)
            m = mat_b[s]
            k = bin_b[s] * 20 + st_b[s] * 5 + dl_b[s]
            ah = plsc.load_gather(aH_v, [m])
            av = plsc.load_gather(aA_v, [m])
            ch = plsc.load_gather(cH_v, [k])
            ca = plsc.load_gather(cA_v, [k])
            lh = jnp.minimum(jnp.maximum(ah + ch, -20.0), 20.0)
            la = jnp.minimum(jnp.maximum(av + ca, -20.0), 20.0)
            return aa + (jnp.exp(lh) + jnp.exp(la)) * dur_b[s]

        return lax.fori_loop(0, _CI // _L, vec, a)

    def int_pair(j, a):
        off = ibase + (2 * j) * _CI
        fire(off + _CI, set1, semB)
        drain(set0, semA)
        a = int_compute(set0, a)
        fire(off + 2 * _CI, set0, semA)
        drain(set1, semB)
        return int_compute(set1, a)

    acc = lax.fori_loop(0, NPAIR - 1, int_pair, acc)
    last = ibase + (2 * NPAIR - 2) * _CI
    fire(last + _CI, set1, semB)
    drain(set0, semA)
    acc = int_compute(set0, acc)
    drain(set1, semB)
    acc = int_compute(set1, acc)

    gbase = wid * PWG
    iv = lax.iota(jnp.int32, _L)

    def goal_chunk(c, a):
        off = gbase + c * _CG
        cps = [
            pltpu.async_copy(gm_h.at[pl.ds(off, _CG)], gm_b, semG),
            pltpu.async_copy(gb_h.at[pl.ds(off, _CG)], gbn_b, semG),
            pltpu.async_copy(gs_h.at[pl.ds(off, _CG)], gst_b, semG),
            pltpu.async_copy(gd_h.at[pl.ds(off, _CG)], gdl_b, semG),
            pltpu.async_copy(gh_h.at[pl.ds(off, _CG)], gh_b, semG),
        ]
        for cp in cps:
            cp.wait()

        def vec(i, aa):
            s = pl.ds(i * _L, _L)
            m = gm_b[s]
            k = gbn_b[s] * 20 + gst_b[s] * 5 + gdl_b[s]
            lh = plsc.load_gather(aH_v, [m]) + plsc.load_gather(cH_v, [k])
            la = plsc.load_gather(aA_v, [m]) + plsc.load_gather(cA_v, [k])
            h = gh_b[s]
            w = jnp.where(iv + (off + i * _L) < _NG_REAL,
                          jnp.float32(1.0), jnp.float32(0.0))
            return aa - (h * lh + (w - h) * la)

        return lax.fori_loop(0, _CG // _L, vec, a)

    acc = lax.fori_loop(0, NCG, goal_chunk, acc)

    accv[...] = acc
    pltpu.sync_copy(accv, out_h.at[pl.ds(wid * _L, _L)])


def kernel(a_H, a_A, a_H_init, a_A_init, b, gamma_H, gamma_A, delta_H,
           delta_A, int_dur, int_match, int_bin, int_state, int_delta,
           goal_match, goal_bin, goal_state, goal_delta, goal_is_home):
    f32 = jnp.float32
    i32 = jnp.int32

    M = a_H.shape[0]
    NI = int_dur.shape[0]
    NG = goal_match.shape[0]
    assert NG == _NG_REAL

    MP = _round_up(M, _NW * _L)
    PWR = MP // _NW
    PW = NI // _NW
    assert PW * _NW == NI and PW % (2 * _CI) == 0
    NPAIR = PW // (2 * _CI)
    NGP = _round_up(NG, _NW * _CG)
    PWG = NGP // _NW
    NCG = PWG // _CG

    zero = jnp.zeros((1,), f32)
    gHf = jnp.concatenate([zero, gamma_H[:1], gamma_H[1:],
                           (gamma_H[0] + gamma_H[1])[None]])
    gAf = jnp.concatenate([zero, gamma_A[:1], gamma_A[1:],
                           (gamma_A[0] + gamma_A[1])[None]])
    dHf = jnp.concatenate([zero, delta_H])
    dAf = jnp.concatenate([zero, delta_A])
    cH = (b[:, None, None] + gHf[None, :, None] + dHf[None, None, :])
    cA = (b[:, None, None] + gAf[None, :, None] + dAf[None, None, :])
    cH = jnp.pad(cH.reshape(-1), (0, 128 - 120))
    cA = jnp.pad(cA.reshape(-1), (0, 128 - 120))
    ro = jnp.pad(jnp.concatenate([b, gamma_H, gamma_A, delta_H, delta_A]),
                 (0, 32 - 18))

    pad_t = (0, MP - M)
    aH_p = jnp.pad(a_H, pad_t)
    aA_p = jnp.pad(a_A, pad_t)
    aHi_p = jnp.pad(a_H_init, pad_t)
    aAi_p = jnp.pad(a_A_init, pad_t)

    pad_g = (0, NGP - NG)
    gm_p = jnp.pad(goal_match.astype(i32), pad_g)
    gb_p = jnp.pad(goal_bin.astype(i32), pad_g)
    gs_p = jnp.pad(goal_state.astype(i32), pad_g)
    gd_p = jnp.pad(goal_delta.astype(i32), pad_g)
    gh_p = jnp.pad(goal_is_home.astype(f32), pad_g)

    mesh = plsc.VectorSubcoreMesh(core_axis_name="c", subcore_axis_name="s",
                                  num_cores=_NC, num_subcores=_NS)
    call = pl.kernel(
        functools.partial(_body, PW, NPAIR, PWG, NCG, PWR),
        out_type=jax.ShapeDtypeStruct((_NW * _L,), f32),
        mesh=mesh,
        compiler_params=pltpu.CompilerParams(needs_layout_passes=False),
        scratch_types=[
            pltpu.VMEM((MP,), f32),
            pltpu.VMEM((MP,), f32),
            pltpu.VMEM((128,), f32),
            pltpu.VMEM((128,), f32),
            pltpu.VMEM((32,), f32),
            pltpu.VMEM((_CI,), f32),
            pltpu.VMEM((_CI,), i32),
            pltpu.VMEM((_CI,), i32),
            pltpu.VMEM((_CI,), i32),
            pltpu.VMEM((_CI,), i32),
            pltpu.VMEM((_CI,), f32),
            pltpu.VMEM((_CI,), i32),
            pltpu.VMEM((_CI,), i32),
            pltpu.VMEM((_CI,), i32),
            pltpu.VMEM((_CI,), i32),
            pltpu.VMEM((_CG,), i32),
            pltpu.VMEM((_CG,), i32),
            pltpu.VMEM((_CG,), i32),
            pltpu.VMEM((_CG,), i32),
            pltpu.VMEM((_CG,), f32),
            pltpu.VMEM((PWR,), f32),
            pltpu.VMEM((PWR,), f32),
            pltpu.VMEM((_L,), f32),
            pltpu.SemaphoreType.DMA,
            pltpu.SemaphoreType.DMA,
            pltpu.SemaphoreType.DMA,
        ],
    )

    out = call(aH_p, aA_p, aHi_p, aAi_p, cH, cA, ro,
               int_dur, int_match.astype(i32), int_bin.astype(i32),
               int_state.astype(i32), int_delta.astype(i32),
               gm_p, gb_p, gs_p, gd_p, gh_p)
    return jnp.sum(out)


# speedup vs baseline: 563.2301x; 1.0186x over previous
import functools

import jax
import jax.numpy as jnp
from jax import lax
from jax.experimental import pallas as pl
from jax.experimental.pallas import tpu as pltpu
from jax.experimental.pallas import tpu_sc as plsc

_SIGMA_A = 1.0
_LAMBDA_REG = 0.01

_NC = 2
_NS = 16
_NW = _NC * _NS
_L = 16

_CI = 2000
_CG = 800
_NG_REAL = 150000


def _round_up(x, m):
    return (x + m - 1) // m * m


def _body(PW, NPAIR, PWG, NCG, PWR,
          aH_h, aA_h, aHi_h, aAi_h, cH_h, cA_h, ro_h,
          dur_h, im_h, ib_h, is_h, id_h,
          gm_h, gb_h, gs_h, gd_h, gh_h,
          out_h,
          aH_v, aA_v, cH_v, cA_v, ro_v,
          dur0, mat0, bin0, st0, dl0,
          dur1, mat1, bin1, st1, dl1,
          gm_b, gbn_b, gst_b, gdl_b, gh_b,
          inibH, inibA, accv, semA, semB, semG):
    wid = lax.axis_index("s") * _NC + lax.axis_index("c")

    ibase = wid * PW
    int_hbm = (dur_h, im_h, ib_h, is_h, id_h)
    set0 = (dur0, mat0, bin0, st0, dl0)
    set1 = (dur1, mat1, bin1, st1, dl1)

    def fire(off, bufs, sem):
        for h, b in zip(int_hbm, bufs):
            pltpu.async_copy(h.at[pl.ds(off, _CI)], b, sem)

    def drain(bufs, sem):
        for h, b in zip(int_hbm, bufs):
            pltpu.make_async_copy(h.at[pl.ds(0, _CI)], b, sem).wait()

    fire(ibase, set0, semA)

    pltpu.sync_copy(aH_h, aH_v)
    pltpu.sync_copy(aA_h, aA_v)
    pltpu.sync_copy(cH_h, cH_v)
    pltpu.sync_copy(cA_h, cA_v)
    pltpu.sync_copy(ro_h, ro_v)

    acc = jnp.zeros((_L,), jnp.float32)

    rbase = wid * PWR
    pltpu.sync_copy(aHi_h.at[pl.ds(rbase, PWR)], inibH)
    pltpu.sync_copy(aAi_h.at[pl.ds(rbase, PWR)], inibA)

    def reg_body(i, a):
        s = pl.ds(rbase + i * _L, _L)
        sl = pl.ds(i * _L, _L)
        dH = aH_v[s] - inibH[sl]
        dA = aA_v[s] - inibA[sl]
        return a + (0.5 / (_SIGMA_A * _SIGMA_A)) * (dH * dH + dA * dA)

    acc = lax.fori_loop(0, PWR // _L, reg_body, acc)

    r0 = ro_v[pl.ds(0, _L)]
    r1 = ro_v[pl.ds(_L, _L)]
    scale = jnp.where(wid == 0, jnp.float32(_LAMBDA_REG), jnp.float32(0.0))
    acc = acc + scale * (r0 * r0 + r1 * r1)

    def int_compute(bufs, a):
        dur_b, mat_b, bin_b, st_b, dl_b = bufs

        def vec(i, aa):
            s = pl.ds(i * _L, _L)
            m = mat_b[s]
            k = bin_b[s] * 20 + st_b[s] * 5 + dl_b[s]
            ah = plsc.load_gather(aH_v, [m])
            av = plsc.load_gather(aA_v, [m])
            ch = plsc.load_gather(cH_v, [k])
            ca = plsc.load_gather(cA_v, [k])
            lh = jnp.minimum(jnp.maximum(ah + ch, -20.0), 20.0)
            la = jnp.minimum(jnp.maximum(av + ca, -20.0), 20.0)
            return aa + (jnp.exp(lh) + jnp.exp(la)) * dur_b[s]

        return lax.fori_loop(0, _CI // _L, vec, a, unroll=5)

    def int_pair(j, a):
        off = ibase + (2 * j) * _CI
        fire(off + _CI, set1, semB)
        drain(set0, semA)
        a = int_compute(set0, a)
        fire(off + 2 * _CI, set0, semA)
        drain(set1, semB)
        return int_compute(set1, a)

    acc = lax.fori_loop(0, NPAIR - 1, int_pair, acc)
    last = ibase + (2 * NPAIR - 2) * _CI
    fire(last + _CI, set1, semB)
    drain(set0, semA)
    acc = int_compute(set0, acc)
    drain(set1, semB)
    acc = int_compute(set1, acc)

    gbase = wid * PWG
    iv = lax.iota(jnp.int32, _L)

    def goal_chunk(c, a):
        off = gbase + c * _CG
        cps = [
            pltpu.async_copy(gm_h.at[pl.ds(off, _CG)], gm_b, semG),
            pltpu.async_copy(gb_h.at[pl.ds(off, _CG)], gbn_b, semG),
            pltpu.async_copy(gs_h.at[pl.ds(off, _CG)], gst_b, semG),
            pltpu.async_copy(gd_h.at[pl.ds(off, _CG)], gdl_b, semG),
            pltpu.async_copy(gh_h.at[pl.ds(off, _CG)], gh_b, semG),
        ]
        for cp in cps:
            cp.wait()

        def vec(i, aa):
            s = pl.ds(i * _L, _L)
            m = gm_b[s]
            k = gbn_b[s] * 20 + gst_b[s] * 5 + gdl_b[s]
            lh = plsc.load_gather(aH_v, [m]) + plsc.load_gather(cH_v, [k])
            la = plsc.load_gather(aA_v, [m]) + plsc.load_gather(cA_v, [k])
            h = gh_b[s]
            w = jnp.where(iv + (off + i * _L) < _NG_REAL,
                          jnp.float32(1.0), jnp.float32(0.0))
            return aa - (h * lh + (w - h) * la)

        return lax.fori_loop(0, _CG // _L, vec, a)

    acc = lax.fori_loop(0, NCG, goal_chunk, acc)

    accv[...] = acc
    pltpu.sync_copy(accv, out_h.at[pl.ds(wid * _L, _L)])


def kernel(a_H, a_A, a_H_init, a_A_init, b, gamma_H, gamma_A, delta_H,
           delta_A, int_dur, int_match, int_bin, int_state, int_delta,
           goal_match, goal_bin, goal_state, goal_delta, goal_is_home):
    f32 = jnp.float32
    i32 = jnp.int32

    M = a_H.shape[0]
    NI = int_dur.shape[0]
    NG = goal_match.shape[0]
    assert NG == _NG_REAL

    MP = _round_up(M, _NW * _L)
    PWR = MP // _NW
    PW = NI // _NW
    assert PW * _NW == NI and PW % (2 * _CI) == 0
    NPAIR = PW // (2 * _CI)
    NGP = _round_up(NG, _NW * _CG)
    PWG = NGP // _NW
    NCG = PWG // _CG

    zero = jnp.zeros((1,), f32)
    gHf = jnp.concatenate([zero, gamma_H[:1], gamma_H[1:],
                           (gamma_H[0] + gamma_H[1])[None]])
    gAf = jnp.concatenate([zero, gamma_A[:1], gamma_A[1:],
                           (gamma_A[0] + gamma_A[1])[None]])
    dHf = jnp.concatenate([zero, delta_H])
    dAf = jnp.concatenate([zero, delta_A])
    cH = (b[:, None, None] + gHf[None, :, None] + dHf[None, None, :])
    cA = (b[:, None, None] + gAf[None, :, None] + dAf[None, None, :])
    cH = jnp.pad(cH.reshape(-1), (0, 128 - 120))
    cA = jnp.pad(cA.reshape(-1), (0, 128 - 120))
    ro = jnp.pad(jnp.concatenate([b, gamma_H, gamma_A, delta_H, delta_A]),
                 (0, 32 - 18))

    pad_t = (0, MP - M)
    aH_p = jnp.pad(a_H, pad_t)
    aA_p = jnp.pad(a_A, pad_t)
    aHi_p = jnp.pad(a_H_init, pad_t)
    aAi_p = jnp.pad(a_A_init, pad_t)

    pad_g = (0, NGP - NG)
    gm_p = jnp.pad(goal_match.astype(i32), pad_g)
    gb_p = jnp.pad(goal_bin.astype(i32), pad_g)
    gs_p = jnp.pad(goal_state.astype(i32), pad_g)
    gd_p = jnp.pad(goal_delta.astype(i32), pad_g)
    gh_p = jnp.pad(goal_is_home.astype(f32), pad_g)

    mesh = plsc.VectorSubcoreMesh(core_axis_name="c", subcore_axis_name="s",
                                  num_cores=_NC, num_subcores=_NS)
    call = pl.kernel(
        functools.partial(_body, PW, NPAIR, PWG, NCG, PWR),
        out_type=jax.ShapeDtypeStruct((_NW * _L,), f32),
        mesh=mesh,
        compiler_params=pltpu.CompilerParams(needs_layout_passes=False),
        scratch_types=[
            pltpu.VMEM((MP,), f32),
            pltpu.VMEM((MP,), f32),
            pltpu.VMEM((128,), f32),
            pltpu.VMEM((128,), f32),
            pltpu.VMEM((32,), f32),
            pltpu.VMEM((_CI,), f32),
            pltpu.VMEM((_CI,), i32),
            pltpu.VMEM((_CI,), i32),
            pltpu.VMEM((_CI,), i32),
            pltpu.VMEM((_CI,), i32),
            pltpu.VMEM((_CI,), f32),
            pltpu.VMEM((_CI,), i32),
            pltpu.VMEM((_CI,), i32),
            pltpu.VMEM((_CI,), i32),
            pltpu.VMEM((_CI,), i32),
            pltpu.VMEM((_CG,), i32),
            pltpu.VMEM((_CG,), i32),
            pltpu.VMEM((_CG,), i32),
            pltpu.VMEM((_CG,), i32),
            pltpu.VMEM((_CG,), f32),
            pltpu.VMEM((PWR,), f32),
            pltpu.VMEM((PWR,), f32),
            pltpu.VMEM((_L,), f32),
            pltpu.SemaphoreType.DMA,
            pltpu.SemaphoreType.DMA,
            pltpu.SemaphoreType.DMA,
        ],
    )

    out = call(aH_p, aA_p, aHi_p, aAi_p, cH, cA, ro,
               int_dur, int_match.astype(i32), int_bin.astype(i32),
               int_state.astype(i32), int_delta.astype(i32),
               gm_p, gb_p, gs_p, gd_p, gh_p)
    return jnp.sum(out)


# speedup vs baseline: 585.5181x; 1.0396x over previous
import functools

import jax
import jax.numpy as jnp
from jax import lax
from jax.experimental import pallas as pl
from jax.experimental.pallas import tpu as pltpu
from jax.experimental.pallas import tpu_sc as plsc

_SIGMA_A = 1.0
_LAMBDA_REG = 0.01

_NC = 2
_NS = 16
_NW = _NC * _NS
_L = 16

_CI = 800
_NB = 5
_CG = 800
_NG_REAL = 150000


def _round_up(x, m):
    return (x + m - 1) // m * m


def _body(PW, NSTEADY, PWG, NCG, PWR,
          aH_h, aA_h, aHi_h, aAi_h, cH_h, cA_h, ro_h,
          dur_h, im_h, ib_h, is_h, id_h,
          gm_h, gb_h, gs_h, gd_h, gh_h,
          out_h,
          aH_v, aA_v, cH_v, cA_v, ro_v,
          *rest):
    int_bufs = [rest[5 * b:5 * b + 5] for b in range(_NB)]
    gm_b, gbn_b, gst_b, gdl_b, gh_b = rest[5 * _NB:5 * _NB + 5]
    inibH, inibA, accv = rest[5 * _NB + 5:5 * _NB + 8]
    sems = rest[5 * _NB + 8:5 * _NB + 8 + _NB]
    semG = rest[5 * _NB + 8 + _NB]

    wid = lax.axis_index("s") * _NC + lax.axis_index("c")

    ibase = wid * PW
    int_hbm = (dur_h, im_h, ib_h, is_h, id_h)

    def fire(off, bufs, sem):
        for h, b in zip(int_hbm, bufs):
            pltpu.async_copy(h.at[pl.ds(off, _CI)], b, sem)

    def drain(bufs, sem):
        for h, b in zip(int_hbm, bufs):
            pltpu.make_async_copy(h.at[pl.ds(0, _CI)], b, sem).wait()

    for b in range(_NB):
        fire(ibase + b * _CI, int_bufs[b], sems[b])

    pltpu.sync_copy(aH_h, aH_v)
    pltpu.sync_copy(aA_h, aA_v)
    pltpu.sync_copy(cH_h, cH_v)
    pltpu.sync_copy(cA_h, cA_v)
    pltpu.sync_copy(ro_h, ro_v)

    acc = jnp.zeros((_L,), jnp.float32)

    rbase = wid * PWR
    pltpu.sync_copy(aHi_h.at[pl.ds(rbase, PWR)], inibH)
    pltpu.sync_copy(aAi_h.at[pl.ds(rbase, PWR)], inibA)

    def reg_body(i, a):
        s = pl.ds(rbase + i * _L, _L)
        sl = pl.ds(i * _L, _L)
        dH = aH_v[s] - inibH[sl]
        dA = aA_v[s] - inibA[sl]
        return a + (0.5 / (_SIGMA_A * _SIGMA_A)) * (dH * dH + dA * dA)

    acc = lax.fori_loop(0, PWR // _L, reg_body, acc)

    r0 = ro_v[pl.ds(0, _L)]
    r1 = ro_v[pl.ds(_L, _L)]
    scale = jnp.where(wid == 0, jnp.float32(_LAMBDA_REG), jnp.float32(0.0))
    acc = acc + scale * (r0 * r0 + r1 * r1)

    def int_compute(bufs, a):
        dur_b, mat_b, bin_b, st_b, dl_b = bufs

        def vec(i, aa):
            s = pl.ds(i * _L, _L)
            m = mat_b[s]
            k = bin_b[s] * 20 + st_b[s] * 5 + dl_b[s]
            ah = plsc.load_gather(aH_v, [m])
            av = plsc.load_gather(aA_v, [m])
            ch = plsc.load_gather(cH_v, [k])
            ca = plsc.load_gather(cA_v, [k])
            lh = jnp.minimum(jnp.maximum(ah + ch, -20.0), 20.0)
            la = jnp.minimum(jnp.maximum(av + ca, -20.0), 20.0)
            return aa + (jnp.exp(lh) + jnp.exp(la)) * dur_b[s]

        return lax.fori_loop(0, _CI // _L, vec, a, unroll=5)

    def int_round(j, a):
        off = ibase + _NB * j * _CI
        for b in range(_NB):
            drain(int_bufs[b], sems[b])
            a = int_compute(int_bufs[b], a)
            fire(off + (_NB + b) * _CI, int_bufs[b], sems[b])
        return a

    acc = lax.fori_loop(0, NSTEADY, int_round, acc)
    for b in range(_NB):
        drain(int_bufs[b], sems[b])
        acc = int_compute(int_bufs[b], acc)

    gbase = wid * PWG
    iv = lax.iota(jnp.int32, _L)

    def goal_chunk(c, a):
        off = gbase + c * _CG
        cps = [
            pltpu.async_copy(gm_h.at[pl.ds(off, _CG)], gm_b, semG),
            pltpu.async_copy(gb_h.at[pl.ds(off, _CG)], gbn_b, semG),
            pltpu.async_copy(gs_h.at[pl.ds(off, _CG)], gst_b, semG),
            pltpu.async_copy(gd_h.at[pl.ds(off, _CG)], gdl_b, semG),
            pltpu.async_copy(gh_h.at[pl.ds(off, _CG)], gh_b, semG),
        ]
        for cp in cps:
            cp.wait()

        def vec(i, aa):
            s = pl.ds(i * _L, _L)
            m = gm_b[s]
            k = gbn_b[s] * 20 + gst_b[s] * 5 + gdl_b[s]
            lh = plsc.load_gather(aH_v, [m]) + plsc.load_gather(cH_v, [k])
            la = plsc.load_gather(aA_v, [m]) + plsc.load_gather(cA_v, [k])
            h = gh_b[s]
            w = jnp.where(iv + (off + i * _L) < _NG_REAL,
                          jnp.float32(1.0), jnp.float32(0.0))
            return aa - (h * lh + (w - h) * la)

        return lax.fori_loop(0, _CG // _L, vec, a)

    acc = lax.fori_loop(0, NCG, goal_chunk, acc)

    accv[...] = acc
    pltpu.sync_copy(accv, out_h.at[pl.ds(wid * _L, _L)])


def kernel(a_H, a_A, a_H_init, a_A_init, b, gamma_H, gamma_A, delta_H,
           delta_A, int_dur, int_match, int_bin, int_state, int_delta,
           goal_match, goal_bin, goal_state, goal_delta, goal_is_home):
    f32 = jnp.float32
    i32 = jnp.int32

    M = a_H.shape[0]
    NI = int_dur.shape[0]
    NG = goal_match.shape[0]
    assert NG == _NG_REAL

    MP = _round_up(M, _NW * _L)
    PWR = MP // _NW
    PW = NI // _NW
    assert PW * _NW == NI and PW % (_NB * _CI) == 0
    NSTEADY = PW // (_NB * _CI) - 1
    NGP = _round_up(NG, _NW * _CG)
    PWG = NGP // _NW
    NCG = PWG // _CG

    zero = jnp.zeros((1,), f32)
    gHf = jnp.concatenate([zero, gamma_H[:1], gamma_H[1:],
                           (gamma_H[0] + gamma_H[1])[None]])
    gAf = jnp.concatenate([zero, gamma_A[:1], gamma_A[1:],
                           (gamma_A[0] + gamma_A[1])[None]])
    dHf = jnp.concatenate([zero, delta_H])
    dAf = jnp.concatenate([zero, delta_A])
    cH = (b[:, None, None] + gHf[None, :, None] + dHf[None, None, :])
    cA = (b[:, None, None] + gAf[None, :, None] + dAf[None, None, :])
    cH = jnp.pad(cH.reshape(-1), (0, 128 - 120))
    cA = jnp.pad(cA.reshape(-1), (0, 128 - 120))
    ro = jnp.pad(jnp.concatenate([b, gamma_H, gamma_A, delta_H, delta_A]),
                 (0, 32 - 18))

    pad_t = (0, MP - M)
    aH_p = jnp.pad(a_H, pad_t)
    aA_p = jnp.pad(a_A, pad_t)
    aHi_p = jnp.pad(a_H_init, pad_t)
    aAi_p = jnp.pad(a_A_init, pad_t)

    pad_g = (0, NGP - NG)
    gm_p = jnp.pad(goal_match.astype(i32), pad_g)
    gb_p = jnp.pad(goal_bin.astype(i32), pad_g)
    gs_p = jnp.pad(goal_state.astype(i32), pad_g)
    gd_p = jnp.pad(goal_delta.astype(i32), pad_g)
    gh_p = jnp.pad(goal_is_home.astype(f32), pad_g)

    mesh = plsc.VectorSubcoreMesh(core_axis_name="c", subcore_axis_name="s",
                                  num_cores=_NC, num_subcores=_NS)
    int_set = [
        pltpu.VMEM((_CI,), f32),
        pltpu.VMEM((_CI,), i32),
        pltpu.VMEM((_CI,), i32),
        pltpu.VMEM((_CI,), i32),
        pltpu.VMEM((_CI,), i32),
    ]
    call = pl.kernel(
        functools.partial(_body, PW, NSTEADY, PWG, NCG, PWR),
        out_type=jax.ShapeDtypeStruct((_NW * _L,), f32),
        mesh=mesh,
        compiler_params=pltpu.CompilerParams(needs_layout_passes=False),
        scratch_types=[
            pltpu.VMEM((MP,), f32),
            pltpu.VMEM((MP,), f32),
            pltpu.VMEM((128,), f32),
            pltpu.VMEM((128,), f32),
            pltpu.VMEM((32,), f32),
        ] + int_set * _NB + [
            pltpu.VMEM((_CG,), i32),
            pltpu.VMEM((_CG,), i32),
            pltpu.VMEM((_CG,), i32),
            pltpu.VMEM((_CG,), i32),
            pltpu.VMEM((_CG,), f32),
            pltpu.VMEM((PWR,), f32),
            pltpu.VMEM((PWR,), f32),
            pltpu.VMEM((_L,), f32),
        ] + [pltpu.SemaphoreType.DMA] * (_NB + 1),
    )

    out = call(aH_p, aA_p, aHi_p, aAi_p, cH, cA, ro,
               int_dur, int_match.astype(i32), int_bin.astype(i32),
               int_state.astype(i32), int_delta.astype(i32),
               gm_p, gb_p, gs_p, gd_p, gh_p)
    return jnp.sum(out)


# speedup vs baseline: 607.5451x; 1.0376x over previous
import functools

import jax
import jax.numpy as jnp
from jax import lax
from jax.experimental import pallas as pl
from jax.experimental.pallas import tpu as pltpu
from jax.experimental.pallas import tpu_sc as plsc

_SIGMA_A = 1.0
_LAMBDA_REG = 0.01

_NC = 2
_NS = 16
_NW = _NC * _NS
_L = 16

_CI = 800
_NB = 5
_CG = 800
_NG_REAL = 150000


def _round_up(x, m):
    return (x + m - 1) // m * m


def _body(PW, NSTEADY, PWG, NCG, PWR,
          aH_h, aA_h, aHi_h, aAi_h, cH_h, cA_h, ro_h,
          dur_h, im_h, ib_h, is_h, id_h,
          gm_h, gb_h, gs_h, gd_h, gh_h,
          out_h,
          aH_v, aA_v, cH_v, cA_v, ro_v,
          *rest):
    int_bufs = [rest[5 * b:5 * b + 5] for b in range(_NB)]
    gm_b, gbn_b, gst_b, gdl_b, gh_b = rest[5 * _NB:5 * _NB + 5]
    inibH, inibA, accv = rest[5 * _NB + 5:5 * _NB + 8]
    sems = rest[5 * _NB + 8:5 * _NB + 8 + _NB]
    semG = rest[5 * _NB + 8 + _NB]

    wid = lax.axis_index("s") * _NC + lax.axis_index("c")

    ibase = wid * PW
    int_hbm = (dur_h, im_h, ib_h, is_h, id_h)

    def fire(off, bufs, sem):
        for h, b in zip(int_hbm, bufs):
            pltpu.async_copy(h.at[pl.ds(off, _CI)], b, sem)

    def drain(bufs, sem):
        for h, b in zip(int_hbm, bufs):
            pltpu.make_async_copy(h.at[pl.ds(0, _CI)], b, sem).wait()

    for b in range(_NB):
        fire(ibase + b * _CI, int_bufs[b], sems[b])

    rbase = wid * PWR
    stage = [
        pltpu.async_copy(aH_h, aH_v, semG),
        pltpu.async_copy(aA_h, aA_v, semG),
        pltpu.async_copy(cH_h, cH_v, semG),
        pltpu.async_copy(cA_h, cA_v, semG),
        pltpu.async_copy(ro_h, ro_v, semG),
        pltpu.async_copy(aHi_h.at[pl.ds(rbase, PWR)], inibH, semG),
        pltpu.async_copy(aAi_h.at[pl.ds(rbase, PWR)], inibA, semG),
    ]
    for cp in stage:
        cp.wait()

    acc = jnp.zeros((_L,), jnp.float32)

    def reg_body(i, a):
        s = pl.ds(rbase + i * _L, _L)
        sl = pl.ds(i * _L, _L)
        dH = aH_v[s] - inibH[sl]
        dA = aA_v[s] - inibA[sl]
        return a + (0.5 / (_SIGMA_A * _SIGMA_A)) * (dH * dH + dA * dA)

    acc = lax.fori_loop(0, PWR // _L, reg_body, acc)

    r0 = ro_v[pl.ds(0, _L)]
    r1 = ro_v[pl.ds(_L, _L)]
    scale = jnp.where(wid == 0, jnp.float32(_LAMBDA_REG), jnp.float32(0.0))
    acc = acc + scale * (r0 * r0 + r1 * r1)

    def int_compute(bufs, a):
        dur_b, mat_b, bin_b, st_b, dl_b = bufs

        def vec(i, aa):
            s = pl.ds(i * _L, _L)
            m = mat_b[s]
            k = bin_b[s] * 20 + st_b[s] * 5 + dl_b[s]
            ah = plsc.load_gather(aH_v, [m])
            av = plsc.load_gather(aA_v, [m])
            ch = plsc.load_gather(cH_v, [k])
            ca = plsc.load_gather(cA_v, [k])
            lh = jnp.minimum(jnp.maximum(ah + ch, -20.0), 20.0)
            la = jnp.minimum(jnp.maximum(av + ca, -20.0), 20.0)
            return aa + (jnp.exp(lh) + jnp.exp(la)) * dur_b[s]

        return lax.fori_loop(0, _CI // _L, vec, a, unroll=5)

    def int_round(j, a):
        off = ibase + _NB * j * _CI
        for b in range(_NB):
            drain(int_bufs[b], sems[b])
            a = int_compute(int_bufs[b], a)
            fire(off + (_NB + b) * _CI, int_bufs[b], sems[b])
        return a

    acc = lax.fori_loop(0, NSTEADY, int_round, acc)
    for b in range(_NB):
        drain(int_bufs[b], sems[b])
        acc = int_compute(int_bufs[b], acc)

    gbase = wid * PWG
    iv = lax.iota(jnp.int32, _L)

    def goal_chunk(c, a):
        off = gbase + c * _CG
        cps = [
            pltpu.async_copy(gm_h.at[pl.ds(off, _CG)], gm_b, semG),
            pltpu.async_copy(gb_h.at[pl.ds(off, _CG)], gbn_b, semG),
            pltpu.async_copy(gs_h.at[pl.ds(off, _CG)], gst_b, semG),
            pltpu.async_copy(gd_h.at[pl.ds(off, _CG)], gdl_b, semG),
            pltpu.async_copy(gh_h.at[pl.ds(off, _CG)], gh_b, semG),
        ]
        for cp in cps:
            cp.wait()

        def vec(i, aa):
            s = pl.ds(i * _L, _L)
            m = gm_b[s]
            k = gbn_b[s] * 20 + gst_b[s] * 5 + gdl_b[s]
            lh = plsc.load_gather(aH_v, [m]) + plsc.load_gather(cH_v, [k])
            la = plsc.load_gather(aA_v, [m]) + plsc.load_gather(cA_v, [k])
            h = gh_b[s]
            w = jnp.where(iv + (off + i * _L) < _NG_REAL,
                          jnp.float32(1.0), jnp.float32(0.0))
            return aa - (h * lh + (w - h) * la)

        return lax.fori_loop(0, _CG // _L, vec, a)

    acc = lax.fori_loop(0, NCG, goal_chunk, acc)

    accv[...] = acc
    pltpu.sync_copy(accv, out_h.at[pl.ds(wid * _L, _L)])


def kernel(a_H, a_A, a_H_init, a_A_init, b, gamma_H, gamma_A, delta_H,
           delta_A, int_dur, int_match, int_bin, int_state, int_delta,
           goal_match, goal_bin, goal_state, goal_delta, goal_is_home):
    f32 = jnp.float32
    i32 = jnp.int32

    M = a_H.shape[0]
    NI = int_dur.shape[0]
    NG = goal_match.shape[0]
    assert NG == _NG_REAL

    MP = _round_up(M, _NW * _L)
    PWR = MP // _NW
    PW = NI // _NW
    assert PW * _NW == NI and PW % (_NB * _CI) == 0
    NSTEADY = PW // (_NB * _CI) - 1
    NGP = _round_up(NG, _NW * _CG)
    PWG = NGP // _NW
    NCG = PWG // _CG

    zero = jnp.zeros((1,), f32)
    gHf = jnp.concatenate([zero, gamma_H[:1], gamma_H[1:],
                           (gamma_H[0] + gamma_H[1])[None]])
    gAf = jnp.concatenate([zero, gamma_A[:1], gamma_A[1:],
                           (gamma_A[0] + gamma_A[1])[None]])
    dHf = jnp.concatenate([zero, delta_H])
    dAf = jnp.concatenate([zero, delta_A])
    cH = (b[:, None, None] + gHf[None, :, None] + dHf[None, None, :])
    cA = (b[:, None, None] + gAf[None, :, None] + dAf[None, None, :])
    cH = jnp.pad(cH.reshape(-1), (0, 128 - 120))
    cA = jnp.pad(cA.reshape(-1), (0, 128 - 120))
    ro = jnp.pad(jnp.concatenate([b, gamma_H, gamma_A, delta_H, delta_A]),
                 (0, 32 - 18))

    pad_t = (0, MP - M)
    aH_p = jnp.pad(a_H, pad_t)
    aA_p = jnp.pad(a_A, pad_t)
    aHi_p = jnp.pad(a_H_init, pad_t)
    aAi_p = jnp.pad(a_A_init, pad_t)

    pad_g = (0, NGP - NG)
    gm_p = jnp.pad(goal_match.astype(i32), pad_g)
    gb_p = jnp.pad(goal_bin.astype(i32), pad_g)
    gs_p = jnp.pad(goal_state.astype(i32), pad_g)
    gd_p = jnp.pad(goal_delta.astype(i32), pad_g)
    gh_p = jnp.pad(goal_is_home.astype(f32), pad_g)

    mesh = plsc.VectorSubcoreMesh(core_axis_name="c", subcore_axis_name="s",
                                  num_cores=_NC, num_subcores=_NS)
    int_set = [
        pltpu.VMEM((_CI,), f32),
        pltpu.VMEM((_CI,), i32),
        pltpu.VMEM((_CI,), i32),
        pltpu.VMEM((_CI,), i32),
        pltpu.VMEM((_CI,), i32),
    ]
    call = pl.kernel(
        functools.partial(_body, PW, NSTEADY, PWG, NCG, PWR),
        out_type=jax.ShapeDtypeStruct((_NW * _L,), f32),
        mesh=mesh,
        compiler_params=pltpu.CompilerParams(needs_layout_passes=False),
        scratch_types=[
            pltpu.VMEM((MP,), f32),
            pltpu.VMEM((MP,), f32),
            pltpu.VMEM((128,), f32),
            pltpu.VMEM((128,), f32),
            pltpu.VMEM((32,), f32),
        ] + int_set * _NB + [
            pltpu.VMEM((_CG,), i32),
            pltpu.VMEM((_CG,), i32),
            pltpu.VMEM((_CG,), i32),
            pltpu.VMEM((_CG,), i32),
            pltpu.VMEM((_CG,), f32),
            pltpu.VMEM((PWR,), f32),
            pltpu.VMEM((PWR,), f32),
            pltpu.VMEM((_L,), f32),
        ] + [pltpu.SemaphoreType.DMA] * (_NB + 1),
    )

    out = call(aH_p, aA_p, aHi_p, aAi_p, cH, cA, ro,
               int_dur, int_match.astype(i32), int_bin.astype(i32),
               int_state.astype(i32), int_delta.astype(i32),
               gm_p, gb_p, gs_p, gd_p, gh_p)
    return jnp.sum(out)


# speedup vs baseline: 625.1440x; 1.0290x over previous
import functools

import jax
import jax.numpy as jnp
from jax import lax
from jax.experimental import pallas as pl
from jax.experimental.pallas import tpu as pltpu
from jax.experimental.pallas import tpu_sc as plsc

_SIGMA_A = 1.0
_LAMBDA_REG = 0.01

_NC = 2
_NS = 16
_NW = _NC * _NS
_L = 16

_CI = 800
_NB = 5
_CG = 800
_NG_REAL = 150000


def _round_up(x, m):
    return (x + m - 1) // m * m


def _body(PW, NSTEADY, PWG, NCG, PWR,
          aH_h, aA_h, aHi_h, aAi_h, cH_h, cA_h, ro_h,
          dur_h, im_h, ib_h, is_h, id_h,
          gm_h, gb_h, gs_h, gd_h, gh_h,
          out_h,
          aH_v, aA_v, cH_v, cA_v, ro_v,
          *rest):
    int_bufs = [rest[5 * b:5 * b + 5] for b in range(_NB)]
    inibH, inibA, accv = rest[5 * _NB:5 * _NB + 3]
    sems = rest[5 * _NB + 3:5 * _NB + 3 + _NB]
    semG = rest[5 * _NB + 3 + _NB]

    wid = lax.axis_index("s") * _NC + lax.axis_index("c")

    ibase = wid * PW
    int_hbm = (dur_h, im_h, ib_h, is_h, id_h)

    def fire(off, bufs, sem):
        for h, b in zip(int_hbm, bufs):
            pltpu.async_copy(h.at[pl.ds(off, _CI)], b, sem)

    def drain(bufs, sem):
        for h, b in zip(int_hbm, bufs):
            pltpu.make_async_copy(h.at[pl.ds(0, _CI)], b, sem).wait()

    for b in range(_NB):
        fire(ibase + b * _CI, int_bufs[b], sems[b])

    rbase = wid * PWR
    stage = [
        pltpu.async_copy(aH_h, aH_v, semG),
        pltpu.async_copy(aA_h, aA_v, semG),
        pltpu.async_copy(cH_h, cH_v, semG),
        pltpu.async_copy(cA_h, cA_v, semG),
        pltpu.async_copy(ro_h, ro_v, semG),
        pltpu.async_copy(aHi_h.at[pl.ds(rbase, PWR)], inibH, semG),
        pltpu.async_copy(aAi_h.at[pl.ds(rbase, PWR)], inibA, semG),
    ]
    for cp in stage:
        cp.wait()

    acc = jnp.zeros((_L,), jnp.float32)

    def reg_body(i, a):
        s = pl.ds(rbase + i * _L, _L)
        sl = pl.ds(i * _L, _L)
        dH = aH_v[s] - inibH[sl]
        dA = aA_v[s] - inibA[sl]
        return a + (0.5 / (_SIGMA_A * _SIGMA_A)) * (dH * dH + dA * dA)

    acc = lax.fori_loop(0, PWR // _L, reg_body, acc)

    r0 = ro_v[pl.ds(0, _L)]
    r1 = ro_v[pl.ds(_L, _L)]
    scale = jnp.where(wid == 0, jnp.float32(_LAMBDA_REG), jnp.float32(0.0))
    acc = acc + scale * (r0 * r0 + r1 * r1)

    def int_compute(bufs, a):
        dur_b, mat_b, bin_b, st_b, dl_b = bufs

        def vec(i, aa):
            s = pl.ds(i * _L, _L)
            m = mat_b[s]
            k = bin_b[s] * 20 + st_b[s] * 5 + dl_b[s]
            ah = plsc.load_gather(aH_v, [m])
            av = plsc.load_gather(aA_v, [m])
            ch = plsc.load_gather(cH_v, [k])
            ca = plsc.load_gather(cA_v, [k])
            lh = jnp.minimum(jnp.maximum(ah + ch, -20.0), 20.0)
            la = jnp.minimum(jnp.maximum(av + ca, -20.0), 20.0)
            return aa + (jnp.exp(lh) + jnp.exp(la)) * dur_b[s]

        return lax.fori_loop(0, _CI // _L, vec, a, unroll=5)

    def int_round(j, a):
        off = ibase + _NB * j * _CI
        for b in range(_NB):
            drain(int_bufs[b], sems[b])
            a = int_compute(int_bufs[b], a)
            fire(off + (_NB + b) * _CI, int_bufs[b], sems[b])
        return a

    acc = lax.fori_loop(0, NSTEADY, int_round, acc)
    for b in range(_NB):
        drain(int_bufs[b], sems[b])
        acc = int_compute(int_bufs[b], acc)

    gbase = wid * PWG
    iv = lax.iota(jnp.int32, _L)
    goal_hbm = (gh_h, gm_h, gb_h, gs_h, gd_h)

    def gfire(off, bufs, sem):
        for h, b in zip(goal_hbm, bufs):
            pltpu.async_copy(h.at[pl.ds(off, _CG)], b, sem)

    def gdrain(bufs, sem):
        for h, b in zip(goal_hbm, bufs):
            pltpu.make_async_copy(h.at[pl.ds(0, _CG)], b, sem).wait()

    def goal_compute(bufs, off, a):
        gh_b, gm_b, gbn_b, gst_b, gdl_b = bufs

        def vec(i, aa):
            s = pl.ds(i * _L, _L)
            m = gm_b[s]
            k = gbn_b[s] * 20 + gst_b[s] * 5 + gdl_b[s]
            lh = plsc.load_gather(aH_v, [m]) + plsc.load_gather(cH_v, [k])
            la = plsc.load_gather(aA_v, [m]) + plsc.load_gather(cA_v, [k])
            h = gh_b[s]
            w = jnp.where(iv + (off + i * _L) < _NG_REAL,
                          jnp.float32(1.0), jnp.float32(0.0))
            return aa - (h * lh + (w - h) * la)

        return lax.fori_loop(0, _CG // _L, vec, a, unroll=5)

    gsetA, gsetB = int_bufs[0], int_bufs[1]
    gfire(gbase, gsetA, sems[0])
    for p in range(NCG // 2):
        off = gbase + 2 * p * _CG
        gfire(off + _CG, gsetB, sems[1])
        gdrain(gsetA, sems[0])
        acc = goal_compute(gsetA, off, acc)
        if p + 1 < NCG // 2:
            gfire(off + 2 * _CG, gsetA, sems[0])
        gdrain(gsetB, sems[1])
        acc = goal_compute(gsetB, off + _CG, acc)

    accv[...] = acc
    pltpu.sync_copy(accv, out_h.at[pl.ds(wid * _L, _L)])


def kernel(a_H, a_A, a_H_init, a_A_init, b, gamma_H, gamma_A, delta_H,
           delta_A, int_dur, int_match, int_bin, int_state, int_delta,
           goal_match, goal_bin, goal_state, goal_delta, goal_is_home):
    f32 = jnp.float32
    i32 = jnp.int32

    M = a_H.shape[0]
    NI = int_dur.shape[0]
    NG = goal_match.shape[0]
    assert NG == _NG_REAL

    MP = _round_up(M, _NW * _L)
    PWR = MP // _NW
    PW = NI // _NW
    assert PW * _NW == NI and PW % (_NB * _CI) == 0
    NSTEADY = PW // (_NB * _CI) - 1
    NGP = _round_up(NG, _NW * _CG)
    PWG = NGP // _NW
    NCG = PWG // _CG

    zero = jnp.zeros((1,), f32)
    gHf = jnp.concatenate([zero, gamma_H[:1], gamma_H[1:],
                           (gamma_H[0] + gamma_H[1])[None]])
    gAf = jnp.concatenate([zero, gamma_A[:1], gamma_A[1:],
                           (gamma_A[0] + gamma_A[1])[None]])
    dHf = jnp.concatenate([zero, delta_H])
    dAf = jnp.concatenate([zero, delta_A])
    cH = (b[:, None, None] + gHf[None, :, None] + dHf[None, None, :])
    cA = (b[:, None, None] + gAf[None, :, None] + dAf[None, None, :])
    cH = jnp.pad(cH.reshape(-1), (0, 128 - 120))
    cA = jnp.pad(cA.reshape(-1), (0, 128 - 120))
    ro = jnp.pad(jnp.concatenate([b, gamma_H, gamma_A, delta_H, delta_A]),
                 (0, 32 - 18))

    pad_t = (0, MP - M)
    aH_p = jnp.pad(a_H, pad_t)
    aA_p = jnp.pad(a_A, pad_t)
    aHi_p = jnp.pad(a_H_init, pad_t)
    aAi_p = jnp.pad(a_A_init, pad_t)

    pad_g = (0, NGP - NG)
    gm_p = jnp.pad(goal_match.astype(i32), pad_g)
    gb_p = jnp.pad(goal_bin.astype(i32), pad_g)
    gs_p = jnp.pad(goal_state.astype(i32), pad_g)
    gd_p = jnp.pad(goal_delta.astype(i32), pad_g)
    gh_p = jnp.pad(goal_is_home.astype(f32), pad_g)

    mesh = plsc.VectorSubcoreMesh(core_axis_name="c", subcore_axis_name="s",
                                  num_cores=_NC, num_subcores=_NS)
    int_set = [
        pltpu.VMEM((_CI,), f32),
        pltpu.VMEM((_CI,), i32),
        pltpu.VMEM((_CI,), i32),
        pltpu.VMEM((_CI,), i32),
        pltpu.VMEM((_CI,), i32),
    ]
    call = pl.kernel(
        functools.partial(_body, PW, NSTEADY, PWG, NCG, PWR),
        out_type=jax.ShapeDtypeStruct((_NW * _L,), f32),
        mesh=mesh,
        compiler_params=pltpu.CompilerParams(needs_layout_passes=False),
        scratch_types=[
            pltpu.VMEM((MP,), f32),
            pltpu.VMEM((MP,), f32),
            pltpu.VMEM((128,), f32),
            pltpu.VMEM((128,), f32),
            pltpu.VMEM((32,), f32),
        ] + int_set * _NB + [
            pltpu.VMEM((PWR,), f32),
            pltpu.VMEM((PWR,), f32),
            pltpu.VMEM((_L,), f32),
        ] + [pltpu.SemaphoreType.DMA] * (_NB + 1),
    )

    out = call(aH_p, aA_p, aHi_p, aAi_p, cH, cA, ro,
               int_dur, int_match.astype(i32), int_bin.astype(i32),
               int_state.astype(i32), int_delta.astype(i32),
               gm_p, gb_p, gs_p, gd_p, gh_p)
    return jnp.sum(out)
